# baseline (speedup 1.0000x reference)
import functools

import jax
import jax.numpy as jnp
from jax import lax
from jax.experimental import pallas as pl
from jax.experimental.pallas import tpu as pltpu
from jax.experimental.pallas import tpu_sc as plsc

VOCAB = 1000000
D = 64

NC = 2
NS = 16
NW = NC * NS

GB = 128
NBUF = 4


@functools.lru_cache(maxsize=None)
def _make_gather(S, B):
    nbb = B // GB
    bb_per_w = nbb // NW
    ng = S * bb_per_w
    mesh = plsc.VectorSubcoreMesh(core_axis_name="c", subcore_axis_name="s")

    @functools.partial(
        pl.kernel,
        mesh=mesh,
        out_type=jax.ShapeDtypeStruct((S, D, B), jnp.float32),
        compiler_params=pltpu.CompilerParams(
            needs_layout_passes=False
        ),
        scratch_types=(
            [
                pltpu.VMEM((S, bb_per_w * GB), jnp.int32),
                pltpu.VMEM((NBUF, GB), jnp.int32),
                pltpu.VMEM((NBUF, GB, 128), jnp.float32),
                pltpu.VMEM((D, GB), jnp.float32),
            ]
            + [pltpu.SemaphoreType.DMA] * NBUF
        ),
    )
    def gather_kernel(idx_hbm, w_hbm, out_hbm, idx_v, pair_v, rows_v, o_v,
                      *sems):
        wid = lax.axis_index("s") * NC + lax.axis_index("c")
        bb0 = wid * bb_per_w

        pltpu.sync_copy(idx_hbm.at[:, pl.ds(bb0 * GB, bb_per_w * GB)], idx_v)
        iota16 = lax.iota(jnp.int32, 16)

        def split(g):
            return g >> 2, g & (bb_per_w - 1)

        def start(g, b):
            s, jb = split(g)
            for j in range(GB // 16):
                v = idx_v[s, pl.ds(jb * GB + j * 16, 16)]
                pair_v[b, pl.ds(j * 16, 16)] = v >> 1
            pltpu.make_async_copy(
                w_hbm.at[pair_v.at[b]], rows_v.at[b], sems[b]
            ).start()

        def drain(g, b):
            s, jb = split(g)
            pltpu.make_async_copy(
                w_hbm.at[pair_v.at[b]], rows_v.at[b], sems[b]
            ).wait()

            def jloop(j, carry):
                v = idx_v[s, pl.ds(jb * GB + j * 16, 16)]
                par = (v & 1) << 6
                row = iota16 + j * 16
                for c in range(D):
                    x = plsc.load_gather(rows_v.at[b], [row, par + c])
                    o_v[c, pl.ds(j * 16, 16)] = x
                return carry

            lax.fori_loop(0, GB // 16, jloop, 0)
            pltpu.sync_copy(
                o_v, out_hbm.at[s, :, pl.ds((bb0 + jb) * GB, GB)]
            )

        for b in range(NBUF):
            start(b, b)

        def outer(o, carry):
            for b in range(NBUF):
                g = o * NBUF + b
                drain(g, b)
                start(g + NBUF, b)
            return carry

        lax.fori_loop(0, ng // NBUF - 1, outer, 0)

        for b in range(NBUF):
            drain((ng // NBUF - 1) * NBUF + b, b)

    return gather_kernel


def kernel(input_, weight):
    n, s = input_.shape
    idx_t = input_.astype(jnp.int32).T
    w_pack = weight.reshape(VOCAB // 2, 128)
    out3 = _make_gather(s, n)(idx_t, w_pack)
    return jnp.transpose(out3, (2, 0, 1))


# speedup vs baseline: 1.7345x; 1.7345x over previous
import functools

import jax
import jax.numpy as jnp
from jax import lax
from jax.experimental import pallas as pl
from jax.experimental.pallas import tpu as pltpu
from jax.experimental.pallas import tpu_sc as plsc

EMBED_DIM = 64

NC = 2
NS = 16
NW = NC * NS

G = 128
NBUF = 4


@functools.lru_cache(maxsize=None)
def _make_gather(B):
    b_per_w = B // NW
    ngrp = b_per_w // G
    nouter = ngrp // NBUF
    mesh = plsc.VectorSubcoreMesh(core_axis_name="c", subcore_axis_name="s")

    @functools.partial(
        pl.kernel,
        mesh=mesh,
        out_type=jax.ShapeDtypeStruct((B, EMBED_DIM), jnp.float32),
        compiler_params=pltpu.CompilerParams(use_tc_tiling_on_sc=False),
        scratch_types=(
            [
                pltpu.VMEM((ngrp, G), jnp.int32),
                pltpu.VMEM((NBUF, G, 128), jnp.float32),
            ]
            + [pltpu.SemaphoreType.DMA] * NBUF
        ),
    )
    def gather_kernel(idx_hbm, w_hbm, out_hbm, idx_v, rows_v, *sems):
        wid = lax.axis_index("s") * NC + lax.axis_index("c")
        base = wid * b_per_w

        pltpu.sync_copy(idx_hbm.at[wid], idx_v)

        def start(g, b):
            pltpu.make_async_copy(
                w_hbm.at[idx_v.at[g]], rows_v.at[b], sems[b]
            ).start()

        def drain(g, b):
            pltpu.make_async_copy(
                w_hbm.at[idx_v.at[g]], rows_v.at[b], sems[b]
            ).wait()
            pltpu.sync_copy(
                rows_v.at[b, :, pl.ds(0, EMBED_DIM)],
                out_hbm.at[pl.ds(base + g * G, G)],
            )

        for b in range(NBUF):
            start(b, b)

        def outer(o, carry):
            for b in range(NBUF):
                g = o * NBUF + b
                drain(g, b)
                start(g + NBUF, b)
            return carry

        lax.fori_loop(0, nouter - 1, outer, 0)

        for b in range(NBUF):
            drain((nouter - 1) * NBUF + b, b)

    return gather_kernel


def kernel(input_, weight):
    n, s = input_.shape
    B = n * s
    idx = input_.astype(jnp.int32).reshape(NW, B // (NW * G), G)
    w_pad = jnp.pad(weight, ((0, 0), (0, 128 - EMBED_DIM)))
    out = _make_gather(B)(idx, w_pad)
    o2 = jnp.transpose(out.reshape(n, s * EMBED_DIM))
    return jnp.transpose(o2.reshape(s, EMBED_DIM, n), (2, 0, 1))
